# baseline (speedup 1.0000x reference)
import functools

import jax
import jax.numpy as jnp
from jax.experimental import pallas as pl
from jax.experimental.pallas import tpu as pltpu
from jax.experimental.pallas import tpu_sc as plsc

D_MODEL = 64
SCALE = 8.0
WINDOW = 256
LANES = 16


def kernel(x, table):
    b, s = x.shape
    n = b * s
    idx = x.reshape(1, n).astype(jnp.int32)
    mesh = plsc.VectorSubcoreMesh(core_axis_name="c", subcore_axis_name="s")

    @functools.partial(
        pl.kernel,
        out_type=jax.ShapeDtypeStruct((n, D_MODEL), table.dtype),
        mesh=mesh,
        compiler_params=pltpu.CompilerParams(use_tc_tiling_on_sc=False),
    )
    def gather_scale(table_hbm, idx_hbm, out_hbm):
        def body(i_vmem, o_vmem):
            pltpu.sync_copy(table_hbm.at[i_vmem.at[0]], o_vmem)

            @pl.loop(0, WINDOW)
            def _row(r):
                @pl.loop(0, D_MODEL, step=LANES)
                def _col(c):
                    o_vmem[r, pl.ds(c, LANES)] = (
                        o_vmem[r, pl.ds(c, LANES)] * SCALE
                    )

        pltpu.emit_pipeline(
            body,
            grid=(n // WINDOW,),
            in_specs=[pl.BlockSpec((1, WINDOW), index_map=lambda i: (0, i))],
            out_specs=[
                pl.BlockSpec((WINDOW, D_MODEL), index_map=lambda i: (i, 0))
            ],
            core_axis_name=("c", "s"),
            dimension_semantics=(pltpu.PARALLEL,),
        )(idx_hbm, out_hbm)

    out = gather_scale(table, idx)
    return out.reshape(b, s, D_MODEL)


# speedup vs baseline: 1.0347x; 1.0347x over previous
import functools

import jax
import jax.numpy as jnp
from jax.experimental import pallas as pl
from jax.experimental.pallas import tpu as pltpu
from jax.experimental.pallas import tpu_sc as plsc

D_MODEL = 64
SCALE = 8.0
WINDOW = 512
LANES = 16
ROW_UNROLL = 4


def kernel(x, table):
    b, s = x.shape
    n = b * s
    idx = x.reshape(1, n).astype(jnp.int32)
    mesh = plsc.VectorSubcoreMesh(core_axis_name="c", subcore_axis_name="s")

    @functools.partial(
        pl.kernel,
        out_type=jax.ShapeDtypeStruct((n, D_MODEL), table.dtype),
        mesh=mesh,
        compiler_params=pltpu.CompilerParams(use_tc_tiling_on_sc=False),
    )
    def gather_scale(table_hbm, idx_hbm, out_hbm):
        def body(i_vmem, o_vmem):
            pltpu.sync_copy(table_hbm.at[i_vmem.at[0]], o_vmem)

            @pl.loop(0, WINDOW, step=ROW_UNROLL)
            def _row(r0):
                for dr in range(ROW_UNROLL):
                    for c in range(0, D_MODEL, LANES):
                        o_vmem[r0 + dr, pl.ds(c, LANES)] = (
                            o_vmem[r0 + dr, pl.ds(c, LANES)] * SCALE
                        )

        pltpu.emit_pipeline(
            body,
            grid=(n // WINDOW,),
            in_specs=[pl.BlockSpec((1, WINDOW), index_map=lambda i: (0, i))],
            out_specs=[
                pl.BlockSpec((WINDOW, D_MODEL), index_map=lambda i: (i, 0))
            ],
            core_axis_name=("c", "s"),
            dimension_semantics=(pltpu.PARALLEL,),
        )(idx_hbm, out_hbm)

    out = gather_scale(table, idx)
    return out.reshape(b, s, D_MODEL)


# speedup vs baseline: 1.4037x; 1.3566x over previous
import functools

import jax
from jax import lax
import jax.numpy as jnp
from jax.experimental import pallas as pl
from jax.experimental.pallas import tpu as pltpu
from jax.experimental.pallas import tpu_sc as plsc

D_MODEL = 64
SCALE = 8.0
W = 256
NBUF = 4
LANES = 16
ROW_UNROLL = 4
NW = 32


def kernel(x, table):
    b, s = x.shape
    n = b * s
    per_w = n // NW
    n_chunks = per_w // W
    idx = x.reshape(n).astype(jnp.int32)
    mesh = plsc.VectorSubcoreMesh(core_axis_name="c", subcore_axis_name="s")

    scratch = (
        [pltpu.VMEM((W,), jnp.int32) for _ in range(NBUF)]
        + [pltpu.VMEM((W, D_MODEL), jnp.float32) for _ in range(NBUF)]
        + [pltpu.SemaphoreType.DMA for _ in range(2 * NBUF)]
    )

    @functools.partial(
        pl.kernel,
        out_type=jax.ShapeDtypeStruct((n, D_MODEL), table.dtype),
        mesh=mesh,
        scratch_types=scratch,
        compiler_params=pltpu.CompilerParams(use_tc_tiling_on_sc=False),
    )
    def gather_scale(table_hbm, idx_hbm, out_hbm, *bufs):
        idx_v = bufs[:NBUF]
        rows_v = bufs[NBUF : 2 * NBUF]
        gsem = bufs[2 * NBUF : 3 * NBUF]
        osem = bufs[3 * NBUF :]

        wid = lax.axis_index("c") * 16 + lax.axis_index("s")
        base = wid * per_w

        def stage_and_fire(g, slot):
            pltpu.sync_copy(idx_hbm.at[pl.ds(base + g * W, W)], idx_v[slot])
            pltpu.async_copy(table_hbm.at[idx_v[slot]], rows_v[slot], gsem[slot])

        def scale_rows(slot):
            @pl.loop(0, W, step=ROW_UNROLL)
            def _row(r0):
                for dr in range(ROW_UNROLL):
                    for c in range(0, D_MODEL, LANES):
                        rows_v[slot][r0 + dr, pl.ds(c, LANES)] = (
                            rows_v[slot][r0 + dr, pl.ds(c, LANES)] * SCALE
                        )

        def wait_gather(slot):
            pltpu.make_async_copy(
                table_hbm.at[idx_v[slot]], rows_v[slot], gsem[slot]
            ).wait()

        def fire_out(g, slot):
            pltpu.async_copy(
                rows_v[slot], out_hbm.at[pl.ds(base + g * W, W)], osem[slot]
            )

        def wait_out(g, slot):
            pltpu.make_async_copy(
                rows_v[slot], out_hbm.at[pl.ds(base + g * W, W)], osem[slot]
            ).wait()

        for slot in range(NBUF):
            stage_and_fire(slot, slot)

        @pl.loop(0, n_chunks - NBUF, step=NBUF)
        def _main(g0):
            for slot in range(NBUF):
                g = g0 + slot
                wait_gather(slot)
                scale_rows(slot)
                fire_out(g, slot)
                wait_out(g, slot)
                stage_and_fire(g + NBUF, slot)

        for g in range(n_chunks - NBUF, n_chunks):
            slot = g % NBUF
            wait_gather(slot)
            scale_rows(slot)
            fire_out(g, slot)
        for g in range(n_chunks - NBUF, n_chunks):
            wait_out(g, g % NBUF)

    out = gather_scale(table, idx)
    return out.reshape(b, s, D_MODEL)


# speedup vs baseline: 1.4326x; 1.0206x over previous
import functools

import jax
from jax import lax
import jax.numpy as jnp
from jax.experimental import pallas as pl
from jax.experimental.pallas import tpu as pltpu
from jax.experimental.pallas import tpu_sc as plsc

D_MODEL = 64
SCALE = 8.0
R = 2
NBUF = 4
LANES = 16
NW = 32


def kernel(x, table):
    x = x.astype(jnp.int32)
    b, s = x.shape
    rows_per_w = b // NW
    n_chunks = rows_per_w // R
    mesh = plsc.VectorSubcoreMesh(core_axis_name="c", subcore_axis_name="s")

    scratch = (
        [pltpu.VMEM((R, s), jnp.int32) for _ in range(NBUF)]
        + [pltpu.VMEM((R, s, D_MODEL), jnp.float32) for _ in range(NBUF)]
        + [pltpu.SemaphoreType.DMA for _ in range(2 * NBUF)]
    )

    @functools.partial(
        pl.kernel,
        out_type=jax.ShapeDtypeStruct((b, s, D_MODEL), table.dtype),
        mesh=mesh,
        scratch_types=scratch,
        compiler_params=pltpu.CompilerParams(use_tc_tiling_on_sc=False),
    )
    def gather_scale(table_hbm, idx_hbm, out_hbm, *bufs):
        idx_v = bufs[:NBUF]
        rows_v = bufs[NBUF : 2 * NBUF]
        gsem = bufs[2 * NBUF : 3 * NBUF]
        osem = bufs[3 * NBUF :]

        wid = lax.axis_index("c") * 16 + lax.axis_index("s")
        base = wid * rows_per_w

        def stage_and_fire(g, slot):
            pltpu.sync_copy(idx_hbm.at[pl.ds(base + g * R, R)], idx_v[slot])
            for r in range(R):
                pltpu.async_copy(
                    table_hbm.at[idx_v[slot].at[r]], rows_v[slot].at[r], gsem[slot]
                )

        def wait_gather(slot):
            for r in range(R):
                pltpu.make_async_copy(
                    table_hbm.at[idx_v[slot].at[r]], rows_v[slot].at[r], gsem[slot]
                ).wait()

        def scale_rows(slot):
            for r in range(R):
                @pl.loop(0, s, step=4)
                def _row(t0):
                    for dt in range(4):
                        for c in range(0, D_MODEL, LANES):
                            rows_v[slot][r, t0 + dt, pl.ds(c, LANES)] = (
                                rows_v[slot][r, t0 + dt, pl.ds(c, LANES)] * SCALE
                            )

        def fire_out(g, slot):
            pltpu.async_copy(
                rows_v[slot], out_hbm.at[pl.ds(base + g * R, R)], osem[slot]
            )

        def wait_out(g, slot):
            pltpu.make_async_copy(
                rows_v[slot], out_hbm.at[pl.ds(base + g * R, R)], osem[slot]
            ).wait()

        for slot in range(NBUF):
            stage_and_fire(slot, slot)

        @pl.loop(0, n_chunks - NBUF, step=NBUF)
        def _main(g0):
            for slot in range(NBUF):
                g = g0 + slot
                wait_gather(slot)
                scale_rows(slot)
                fire_out(g, slot)
                wait_out(g, slot)
                stage_and_fire(g + NBUF, slot)

        for g in range(n_chunks - NBUF, n_chunks):
            slot = g % NBUF
            wait_gather(slot)
            scale_rows(slot)
            fire_out(g, slot)
        for g in range(n_chunks - NBUF, n_chunks):
            wait_out(g, g % NBUF)

    return gather_scale(table, x)


# speedup vs baseline: 1.4610x; 1.0198x over previous
import functools

import jax
from jax import lax
import jax.numpy as jnp
from jax.experimental import pallas as pl
from jax.experimental.pallas import tpu as pltpu
from jax.experimental.pallas import tpu_sc as plsc

D_MODEL = 64
SCALE = 8.0
NRING = 8
GDEPTH = 4
LANES = 16
NW = 32


def kernel(x, table):
    x = x.astype(jnp.int32)
    b, s = x.shape
    rows_per_w = b // NW
    n = rows_per_w
    mesh = plsc.VectorSubcoreMesh(core_axis_name="c", subcore_axis_name="s")

    scratch = (
        [pltpu.VMEM((rows_per_w, s), jnp.int32)]
        + [pltpu.VMEM((s, D_MODEL), jnp.float32) for _ in range(NRING)]
        + [pltpu.SemaphoreType.DMA for _ in range(2 * NRING)]
    )

    @functools.partial(
        pl.kernel,
        out_type=jax.ShapeDtypeStruct((b, s, D_MODEL), table.dtype),
        mesh=mesh,
        scratch_types=scratch,
        compiler_params=pltpu.CompilerParams(use_tc_tiling_on_sc=False),
    )
    def gather_scale(table_hbm, idx_hbm, out_hbm, *bufs):
        idx_all = bufs[0]
        rows_v = bufs[1 : 1 + NRING]
        gsem = bufs[1 + NRING : 1 + 2 * NRING]
        osem = bufs[1 + 2 * NRING :]

        wid = lax.axis_index("c") * 16 + lax.axis_index("s")
        base = wid * rows_per_w

        def fire_gather(g, slot):
            pltpu.async_copy(table_hbm.at[idx_all.at[g]], rows_v[slot], gsem[slot])

        def wait_gather(g, slot):
            pltpu.make_async_copy(
                table_hbm.at[idx_all.at[g]], rows_v[slot], gsem[slot]
            ).wait()

        def scale_rows(slot):
            @pl.loop(0, s, step=4)
            def _row(t0):
                for dt in range(4):
                    for c in range(0, D_MODEL, LANES):
                        rows_v[slot][t0 + dt, pl.ds(c, LANES)] = (
                            rows_v[slot][t0 + dt, pl.ds(c, LANES)] * SCALE
                        )

        def fire_out(g, slot):
            pltpu.async_copy(rows_v[slot], out_hbm.at[base + g], osem[slot])

        def wait_out(g, slot):
            pltpu.make_async_copy(
                rows_v[slot], out_hbm.at[base + g], osem[slot]
            ).wait()

        pltpu.sync_copy(idx_hbm.at[pl.ds(base, rows_per_w)], idx_all)

        for g in range(GDEPTH):
            fire_gather(g, g)

        for g in range(GDEPTH):
            wait_gather(g, g)
            scale_rows(g)
            fire_out(g, g)
            fire_gather(g + GDEPTH, g + GDEPTH)

        @pl.loop(GDEPTH, n - GDEPTH, step=NRING)
        def _main(g0):
            for k in range(NRING):
                g = g0 + k
                slot = (GDEPTH + k) % NRING
                wait_gather(g, slot)
                scale_rows(slot)
                fire_out(g, slot)
                wait_out(g - GDEPTH, k)
                fire_gather(g + GDEPTH, k)

        for k in range(GDEPTH):
            g = n - GDEPTH + k
            slot = (GDEPTH + k) % NRING
            wait_gather(g, slot)
            scale_rows(slot)
            fire_out(g, slot)
        for g in range(n - NRING, n):
            wait_out(g, g % NRING)

    return gather_scale(table, x)


# speedup vs baseline: 1.4635x; 1.0017x over previous
import functools

import jax
from jax import lax
import jax.numpy as jnp
from jax.experimental import pallas as pl
from jax.experimental.pallas import tpu as pltpu
from jax.experimental.pallas import tpu_sc as plsc

D_MODEL = 64
SCALE = 8.0
NRING = 8
GDEPTH = 4
LANES = 16
NW = 32


def kernel(x, table):
    x = x.astype(jnp.int32)
    b, s = x.shape
    rows_per_w = b // NW
    n = rows_per_w
    mesh = plsc.VectorSubcoreMesh(core_axis_name="c", subcore_axis_name="s")

    scratch = (
        [pltpu.VMEM((rows_per_w, s), jnp.int32)]
        + [pltpu.VMEM((s, D_MODEL), jnp.float32) for _ in range(NRING)]
        + [pltpu.SemaphoreType.DMA for _ in range(2 * NRING)]
    )

    @functools.partial(
        pl.kernel,
        out_type=jax.ShapeDtypeStruct((b * s, D_MODEL), table.dtype),
        mesh=mesh,
        scratch_types=scratch,
        compiler_params=pltpu.CompilerParams(use_tc_tiling_on_sc=False),
    )
    def gather_scale(table_hbm, idx_hbm, out_hbm, *bufs):
        idx_all = bufs[0]
        rows_v = bufs[1 : 1 + NRING]
        gsem = bufs[1 + NRING : 1 + 2 * NRING]
        osem = bufs[1 + 2 * NRING :]

        wid = lax.axis_index("c") * 16 + lax.axis_index("s")
        base = wid * rows_per_w

        def fire_gather(g, slot):
            pltpu.async_copy(table_hbm.at[idx_all.at[g]], rows_v[slot], gsem[slot])

        def wait_gather(g, slot):
            pltpu.make_async_copy(
                table_hbm.at[idx_all.at[g]], rows_v[slot], gsem[slot]
            ).wait()

        def scale_rows(slot):
            @pl.loop(0, s, step=4)
            def _row(t0):
                for dt in range(4):
                    for c in range(0, D_MODEL, LANES):
                        rows_v[slot][t0 + dt, pl.ds(c, LANES)] = (
                            rows_v[slot][t0 + dt, pl.ds(c, LANES)] * SCALE
                        )

        def fire_out(g, slot):
            pltpu.async_copy(
                rows_v[slot], out_hbm.at[pl.ds((base + g) * s, s)], osem[slot]
            )

        def wait_out(g, slot):
            pltpu.make_async_copy(
                rows_v[slot], out_hbm.at[pl.ds((base + g) * s, s)], osem[slot]
            ).wait()

        pltpu.sync_copy(idx_hbm.at[pl.ds(base, rows_per_w)], idx_all)

        for g in range(GDEPTH):
            fire_gather(g, g)

        for g in range(GDEPTH):
            wait_gather(g, g)
            scale_rows(g)
            fire_out(g, g)
            fire_gather(g + GDEPTH, g + GDEPTH)

        @pl.loop(GDEPTH, n - GDEPTH, step=NRING)
        def _main(g0):
            for k in range(NRING):
                g = g0 + k
                slot = (GDEPTH + k) % NRING
                wait_gather(g, slot)
                scale_rows(slot)
                fire_out(g, slot)
                wait_out(g - GDEPTH, k)
                fire_gather(g + GDEPTH, k)

        for k in range(GDEPTH):
            g = n - GDEPTH + k
            slot = (GDEPTH + k) % NRING
            wait_gather(g, slot)
            scale_rows(slot)
            fire_out(g, slot)
        for g in range(n - NRING, n):
            wait_out(g, g % NRING)

    return gather_scale(table, x).reshape(b, s, D_MODEL)
